# Initial kernel scaffold; baseline (speedup 1.0000x reference)
import jax
import jax.numpy as jnp
from jax.experimental import pallas as pl


def kernel(x, index):
    raise NotImplementedError("write your pallas kernel here")


# speedup vs baseline: 1.7938x; 1.7938x over previous
import functools

import jax
import jax.numpy as jnp
from jax import lax
from jax.experimental import pallas as pl
from jax.experimental.pallas import tpu as pltpu
from jax.experimental.pallas import tpu_sc as plsc

NUM_ROWS = 160000
NUM_COLS = 256
N_SEG = 10000
N_SEG_PAD = 10240

NC = 2
NS = 16
L = 16

COLS_PER_SC = NUM_COLS // NC
COL_TILE = 64
N_PHASES = COLS_PER_SC // COL_TILE
ROWS_PER_TILE = NUM_ROWS // NS
CHUNK = 80
N_CHUNKS = ROWS_PER_TILE // CHUNK
SEG_PER_TILE = N_SEG_PAD // NS


def _body(x_hbm, idx_hbm, out_hbm,
          idx_v, rows_v, ones_v, sums_v, cnt_v, acc_sh, cntacc_sh):
    c = lax.axis_index("c")
    s = lax.axis_index("s")
    seg0 = s * SEG_PER_TILE
    row_base = s * ROWS_PER_TILE

    zeros16 = jnp.zeros((L,), jnp.float32)
    ones16 = jnp.ones((L,), jnp.float32)

    def ones_body(i, _):
        ones_v[i, :] = ones16
        return 0

    lax.fori_loop(0, CHUNK, ones_body, 0)

    for ph in range(N_PHASES):
        col0 = c * COLS_PER_SC + ph * COL_TILE

        def zero_body(i, _):
            if ph == 0:
                cnt_v[i, :] = zeros16
            for j in range(COL_TILE // L):
                sums_v[i, pl.ds(j * L, L)] = zeros16
            return 0

        lax.fori_loop(0, SEG_PER_TILE, zero_body, 0)
        pltpu.sync_copy(sums_v, acc_sh.at[pl.ds(seg0, SEG_PER_TILE)])
        if ph == 0:
            pltpu.sync_copy(cnt_v, cntacc_sh.at[pl.ds(seg0, SEG_PER_TILE)])

        plsc.subcore_barrier()

        def chunk_body(k, _):
            base = row_base + k * CHUNK
            pltpu.sync_copy(idx_hbm.at[pl.ds(base, CHUNK)], idx_v)
            pltpu.sync_copy(x_hbm.at[pl.ds(base, CHUNK), pl.ds(col0, COL_TILE)],
                            rows_v)
            pltpu.sync_copy(rows_v, acc_sh.at[idx_v], add=True)
            if ph == 0:
                pltpu.sync_copy(ones_v, cntacc_sh.at[idx_v], add=True)
            return 0

        lax.fori_loop(0, N_CHUNKS, chunk_body, 0)

        plsc.subcore_barrier()

        pltpu.sync_copy(acc_sh.at[pl.ds(seg0, SEG_PER_TILE)], sums_v)
        if ph == 0:
            pltpu.sync_copy(cntacc_sh.at[pl.ds(seg0, SEG_PER_TILE)], cnt_v)

        def div_body(i, _):
            inv = ones16 / jnp.maximum(cnt_v[i, :], ones16)
            for j in range(COL_TILE // L):
                sums_v[i, pl.ds(j * L, L)] = sums_v[i, pl.ds(j * L, L)] * inv
            return 0

        lax.fori_loop(0, SEG_PER_TILE, div_body, 0)
        pltpu.sync_copy(sums_v,
                        out_hbm.at[pl.ds(seg0, SEG_PER_TILE),
                                   pl.ds(col0, COL_TILE)])


@jax.jit
def _mean_pool(x, index):
    run = pl.kernel(
        _body,
        out_type=jax.ShapeDtypeStruct((N_SEG_PAD, NUM_COLS), jnp.float32),
        mesh=plsc.VectorSubcoreMesh(core_axis_name="c", subcore_axis_name="s"),
        compiler_params=pltpu.CompilerParams(use_tc_tiling_on_sc=False),
        scratch_types=[
            pltpu.VMEM((CHUNK,), jnp.int32),
            pltpu.VMEM((CHUNK, COL_TILE), jnp.float32),
            pltpu.VMEM((CHUNK, L), jnp.float32),
            pltpu.VMEM((SEG_PER_TILE, COL_TILE), jnp.float32),
            pltpu.VMEM((SEG_PER_TILE, L), jnp.float32),
            pltpu.VMEM_SHARED((N_SEG_PAD, COL_TILE), jnp.float32),
            pltpu.VMEM_SHARED((N_SEG_PAD, L), jnp.float32),
        ],
    )
    return run(x, index)[:N_SEG]


def kernel(x, index):
    return _mean_pool(x, index.astype(jnp.int32))


# speedup vs baseline: 2.6659x; 1.4862x over previous
import jax
import jax.numpy as jnp
from jax import lax
from jax.experimental import pallas as pl
from jax.experimental.pallas import tpu as pltpu
from jax.experimental.pallas import tpu_sc as plsc

NUM_ROWS = 160000
NUM_COLS = 256
N_SEG = 10000
N_SEG_PAD = 10240

NC = 2
NS = 16
L = 16

COLS_PER_SC = NUM_COLS // NC
COL_TILE = 64
N_PHASES = COLS_PER_SC // COL_TILE
ROWS_PER_TILE = NUM_ROWS // NS
CHUNK = 80
N_CHUNKS = ROWS_PER_TILE // CHUNK
N_PAIRS = (N_CHUNKS - 1) // 2
SEG_PER_TILE = N_SEG_PAD // NS
LAST_SEGS = N_SEG - (NS - 1) * SEG_PER_TILE


def _body(x_hbm, idx2d_hbm, out_hbm,
          idxb0, idxb1, rows0, rows1, ones_v, sums_v, cnt_v,
          acc_sh, cntacc_sh,
          sem_l0, sem_l1, sem_i0, sem_i1, sem_s0, sem_s1, sem_o0, sem_o1):
    c = lax.axis_index("c")
    s = lax.axis_index("s")
    seg0 = s * SEG_PER_TILE
    row_base = s * ROWS_PER_TILE

    zeros16 = jnp.zeros((L,), jnp.float32)
    ones16 = jnp.ones((L,), jnp.float32)

    def ones_body(i, _):
        ones_v[i, :] = ones16
        return 0

    lax.fori_loop(0, CHUNK, ones_body, 0)

    for ph in range(N_PHASES):
        col0 = c * COLS_PER_SC + ph * COL_TILE
        rows = (rows0, rows1)
        idxb = (idxb0, idxb1)
        sem_l = (sem_l0, sem_l1)
        sem_i = (sem_i0, sem_i1)
        sem_s = (sem_s0, sem_s1)
        sem_o = (sem_o0, sem_o1)

        def x_src(k):
            return x_hbm.at[pl.ds(row_base + k * CHUNK, CHUNK),
                            pl.ds(col0, COL_TILE)]

        def start_load(k, p):
            pltpu.async_copy(x_src(k), rows[p], sem_l[p])
            pltpu.async_copy(idx2d_hbm.at[s * N_CHUNKS + k], idxb[p],
                             sem_i[p])

        def wait_load(p):
            pltpu.make_async_copy(x_src(0), rows[p], sem_l[p]).wait()
            pltpu.make_async_copy(idx2d_hbm.at[0], idxb[p], sem_i[p]).wait()

        def start_scatter(k, p):
            pltpu.async_copy(rows[p], acc_sh.at[idxb[p]], sem_s[p],
                             add=True)
            if ph == 0:
                pltpu.async_copy(ones_v, cntacc_sh.at[idxb[p]],
                                 sem_o[p], add=True)

        def wait_scatter(p):
            pltpu.make_async_copy(rows[p], acc_sh.at[idxb[p]],
                                  sem_s[p]).wait()
            if ph == 0:
                pltpu.make_async_copy(ones_v, cntacc_sh.at[idxb[p]],
                                      sem_o[p]).wait()

        def zero_body(i, _):
            if ph == 0:
                cnt_v[i, :] = zeros16
            for j in range(COL_TILE // L):
                sums_v[i, pl.ds(j * L, L)] = zeros16
            return 0

        lax.fori_loop(0, SEG_PER_TILE, zero_body, 0)
        pltpu.sync_copy(sums_v, acc_sh.at[pl.ds(seg0, SEG_PER_TILE)])
        if ph == 0:
            pltpu.sync_copy(cnt_v, cntacc_sh.at[pl.ds(seg0, SEG_PER_TILE)])

        plsc.subcore_barrier()

        start_load(0, 0)

        def pair_body(j, _):
            k0 = 2 * j
            wait_load(0)

            @pl.when(j > 0)
            def _():
                wait_scatter(1)

            start_load(k0 + 1, 1)
            start_scatter(k0, 0)
            wait_load(1)
            wait_scatter(0)
            start_load(k0 + 2, 0)
            start_scatter(k0 + 1, 1)
            return 0

        lax.fori_loop(0, N_PAIRS, pair_body, 0)

        wait_load(0)
        wait_scatter(1)
        start_scatter(N_CHUNKS - 1, 0)
        wait_scatter(0)

        plsc.subcore_barrier()

        pltpu.sync_copy(acc_sh.at[pl.ds(seg0, SEG_PER_TILE)], sums_v)
        if ph == 0:
            pltpu.sync_copy(cntacc_sh.at[pl.ds(seg0, SEG_PER_TILE)], cnt_v)

            def inv_body(i, _):
                cnt_v[i, :] = ones16 / jnp.maximum(cnt_v[i, :], ones16)
                return 0

            lax.fori_loop(0, SEG_PER_TILE, inv_body, 0)

        def div_body(i, _):
            inv = cnt_v[i, :]
            for j in range(COL_TILE // L):
                sums_v[i, pl.ds(j * L, L)] = sums_v[i, pl.ds(j * L, L)] * inv
            return 0

        lax.fori_loop(0, SEG_PER_TILE, div_body, 0)

        @pl.when(s < NS - 1)
        def _():
            pltpu.sync_copy(sums_v,
                            out_hbm.at[pl.ds(seg0, SEG_PER_TILE),
                                       pl.ds(col0, COL_TILE)])

        @pl.when(s == NS - 1)
        def _():
            pltpu.sync_copy(sums_v.at[pl.ds(0, LAST_SEGS)],
                            out_hbm.at[pl.ds(seg0, LAST_SEGS),
                                       pl.ds(col0, COL_TILE)])


@jax.jit
def _mean_pool(x, idx2d):
    run = pl.kernel(
        _body,
        out_type=jax.ShapeDtypeStruct((N_SEG, NUM_COLS), jnp.float32),
        mesh=plsc.VectorSubcoreMesh(core_axis_name="c", subcore_axis_name="s"),
        compiler_params=pltpu.CompilerParams(use_tc_tiling_on_sc=False),
        scratch_types=[
            pltpu.VMEM((CHUNK,), jnp.int32),
            pltpu.VMEM((CHUNK,), jnp.int32),
            pltpu.VMEM((CHUNK, COL_TILE), jnp.float32),
            pltpu.VMEM((CHUNK, COL_TILE), jnp.float32),
            pltpu.VMEM((CHUNK, L), jnp.float32),
            pltpu.VMEM((SEG_PER_TILE, COL_TILE), jnp.float32),
            pltpu.VMEM((SEG_PER_TILE, L), jnp.float32),
            pltpu.VMEM_SHARED((N_SEG_PAD, COL_TILE), jnp.float32),
            pltpu.VMEM_SHARED((N_SEG_PAD, L), jnp.float32),
            pltpu.SemaphoreType.DMA,
            pltpu.SemaphoreType.DMA,
            pltpu.SemaphoreType.DMA,
            pltpu.SemaphoreType.DMA,
            pltpu.SemaphoreType.DMA,
            pltpu.SemaphoreType.DMA,
            pltpu.SemaphoreType.DMA,
            pltpu.SemaphoreType.DMA,
        ],
    )
    return run(x, idx2d)


def kernel(x, index):
    idx2d = index.astype(jnp.int32).reshape(NUM_ROWS // CHUNK, CHUNK)
    return _mean_pool(x, idx2d)


# speedup vs baseline: 3.6322x; 1.3625x over previous
import jax
import jax.numpy as jnp
from jax import lax
from jax.experimental import pallas as pl
from jax.experimental.pallas import tpu as pltpu
from jax.experimental.pallas import tpu_sc as plsc

NUM_ROWS = 160000
NUM_COLS = 256
N_SEG = 10000
N_SEG_PAD = 10240

NC = 2
NS = 16
L = 16

COLS_PER_SC = NUM_COLS // NC
ROWS_PER_TILE = NUM_ROWS // NS
CHUNK = 64
N_MAIN = ROWS_PER_TILE // CHUNK
TAIL = ROWS_PER_TILE - N_MAIN * CHUNK
NBUF = 4
N_GROUPS = N_MAIN // NBUF
LOAD_AHEAD = 2
SEG_PER_TILE = N_SEG_PAD // NS
STRIP = CHUNK
N_STRIPS = SEG_PER_TILE // STRIP
LAST_SEGS = N_SEG - (NS - 1) * SEG_PER_TILE
LAST_FULL_STRIPS = LAST_SEGS // STRIP
LAST_PART = LAST_SEGS - LAST_FULL_STRIPS * STRIP


def _body(x_hbm, idx_hbm, out_hbm,
          idxb, rowsb, idxt_v, tail_v, ones_v, cnt_v,
          acc_sh, cntacc_sh, sems, out_sems):
    c = lax.axis_index("c")
    s = lax.axis_index("s")
    seg0 = s * SEG_PER_TILE
    row_base = s * ROWS_PER_TILE
    col0 = c * COLS_PER_SC

    zeros16 = jnp.zeros((L,), jnp.float32)
    ones16 = jnp.ones((L,), jnp.float32)

    def ones_body(i, _):
        ones_v[i, :] = ones16
        return 0

    lax.fori_loop(0, CHUNK, ones_body, 0)

    def x_src(k, n=CHUNK):
        return x_hbm.at[pl.ds(row_base + k * CHUNK, n),
                        pl.ds(col0, COLS_PER_SC)]

    def start_load(k, b):
        pltpu.async_copy(x_src(k), rowsb[b], sems[b])
        pltpu.async_copy(idx_hbm.at[pl.ds(row_base + k * CHUNK, CHUNK)],
                         idxb[b], sems[b])

    def wait_load(b):
        pltpu.make_async_copy(x_src(0), rowsb[b], sems[b]).wait()
        pltpu.make_async_copy(idx_hbm.at[pl.ds(row_base, CHUNK)],
                              idxb[b], sems[b]).wait()

    def start_scatter(k, b):
        pltpu.async_copy(rowsb[b], acc_sh.at[idxb[b]], sems[NBUF + b],
                         add=True)
        pltpu.async_copy(ones_v, cntacc_sh.at[idxb[b]], sems[NBUF + b],
                         add=True)

    def wait_scatter(b):
        pltpu.make_async_copy(rowsb[b], acc_sh.at[idxb[b]],
                              sems[NBUF + b]).wait()
        pltpu.make_async_copy(ones_v, cntacc_sh.at[idxb[b]],
                              sems[NBUF + b]).wait()

    def zero_body(i, _):
        for j in range(COLS_PER_SC // L):
            rowsb[0][i, pl.ds(j * L, L)] = zeros16
        cnt_v[i, :] = zeros16
        return 0

    lax.fori_loop(0, CHUNK, zero_body, 0)
    for st in range(N_STRIPS):
        pltpu.sync_copy(rowsb[0], acc_sh.at[pl.ds(seg0 + st * STRIP, STRIP)])
        pltpu.sync_copy(cnt_v, cntacc_sh.at[pl.ds(seg0 + st * STRIP, STRIP)])

    for b in range(LOAD_AHEAD):
        start_load(b, b)

    plsc.subcore_barrier()

    def slot(j, b, drain, load):
        wait_load(b)
        start_scatter(j, b)
        if drain:
            wait_scatter((b + LOAD_AHEAD) % NBUF)
        if load:
            start_load(j + LOAD_AHEAD, (b + LOAD_AHEAD) % NBUF)

    for b in range(NBUF):
        slot(b, b, b >= NBUF - LOAD_AHEAD, True)

    def group_body(g, _):
        j0 = g * NBUF
        for b in range(NBUF):
            slot(j0 + b, b, True, True)
        return 0

    lax.fori_loop(1, N_GROUPS - 1, group_body, 0)

    j0 = (N_GROUPS - 1) * NBUF
    for b in range(NBUF):
        slot(j0 + b, b, True, b < NBUF - LOAD_AHEAD)
    for b in range(LOAD_AHEAD, NBUF):
        wait_scatter(b)

    pltpu.sync_copy(idx_hbm.at[pl.ds(row_base + N_MAIN * CHUNK, TAIL)],
                    idxt_v)
    pltpu.sync_copy(x_src(N_MAIN, TAIL), tail_v)
    pltpu.sync_copy(tail_v, acc_sh.at[idxt_v], add=True)
    pltpu.sync_copy(ones_v.at[pl.ds(0, TAIL)], cntacc_sh.at[idxt_v], add=True)

    plsc.subcore_barrier()

    def out_dst(st, n=STRIP):
        return out_hbm.at[pl.ds(seg0 + st * STRIP, n),
                          pl.ds(col0, COLS_PER_SC)]

    def finalize_strip(st, b):
        pltpu.sync_copy(acc_sh.at[pl.ds(seg0 + st * STRIP, STRIP)], rowsb[b])
        pltpu.sync_copy(cntacc_sh.at[pl.ds(seg0 + st * STRIP, STRIP)], cnt_v)

        def div_body(i, _):
            inv = ones16 / jnp.maximum(cnt_v[i, :], ones16)
            for j in range(COLS_PER_SC // L):
                rowsb[b][i, pl.ds(j * L, L)] = (
                    rowsb[b][i, pl.ds(j * L, L)] * inv)
            return 0

        lax.fori_loop(0, STRIP, div_body, 0)

    def wait_out_full(st_done, b):
        pltpu.make_async_copy(rowsb[b], out_dst(st_done), out_sems[b]).wait()

    def wait_out_part(st_done, b):
        pltpu.make_async_copy(rowsb[b].at[pl.ds(0, LAST_PART)],
                              out_dst(st_done, LAST_PART),
                              out_sems[b]).wait()

    for st in range(N_STRIPS):
        b = st % 2
        if st >= 2:
            st_done = st - 2
            if st_done < LAST_FULL_STRIPS:
                wait_out_full(st_done, b)
            else:
                @pl.when(s < NS - 1)
                def _():
                    wait_out_full(st_done, b)

                if st_done == LAST_FULL_STRIPS:
                    @pl.when(s == NS - 1)
                    def _():
                        wait_out_part(st_done, b)

        finalize_strip(st, b)

        if st < LAST_FULL_STRIPS:
            pltpu.async_copy(rowsb[b], out_dst(st), out_sems[b])
        else:
            @pl.when(s < NS - 1)
            def _():
                pltpu.async_copy(rowsb[b], out_dst(st), out_sems[b])

            if st == LAST_FULL_STRIPS:
                @pl.when(s == NS - 1)
                def _():
                    pltpu.async_copy(rowsb[b].at[pl.ds(0, LAST_PART)],
                                     out_dst(st, LAST_PART), out_sems[b])

    @pl.when(s < NS - 1)
    def _():
        for st_done in (N_STRIPS - 2, N_STRIPS - 1):
            wait_out_full(st_done, st_done % 2)


@jax.jit
def _mean_pool(x, index):
    run = pl.kernel(
        _body,
        out_type=jax.ShapeDtypeStruct((N_SEG, NUM_COLS), jnp.float32),
        mesh=plsc.VectorSubcoreMesh(core_axis_name="c", subcore_axis_name="s"),
        compiler_params=pltpu.CompilerParams(use_tc_tiling_on_sc=False),
        scratch_types=[
            [pltpu.VMEM((CHUNK,), jnp.int32) for _ in range(NBUF)],
            [pltpu.VMEM((CHUNK, COLS_PER_SC), jnp.float32)
             for _ in range(NBUF)],
            pltpu.VMEM((TAIL,), jnp.int32),
            pltpu.VMEM((TAIL, COLS_PER_SC), jnp.float32),
            pltpu.VMEM((CHUNK, L), jnp.float32),
            pltpu.VMEM((STRIP, L), jnp.float32),
            pltpu.VMEM_SHARED((N_SEG_PAD, COLS_PER_SC), jnp.float32),
            pltpu.VMEM_SHARED((N_SEG_PAD, L), jnp.float32),
            [pltpu.SemaphoreType.DMA for _ in range(2 * NBUF)],
            [pltpu.SemaphoreType.DMA for _ in range(2)],
        ],
    )
    return run(x, index)


def kernel(x, index):
    return _mean_pool(x, index.astype(jnp.int32))


# speedup vs baseline: 6.6106x; 1.8200x over previous
import jax
import jax.numpy as jnp
from jax import lax
from jax.experimental import pallas as pl
from jax.experimental.pallas import tpu as pltpu
from jax.experimental.pallas import tpu_sc as plsc

NUM_ROWS = 160000
NUM_COLS = 256
N_SEG = 10000
N_SEG_PAD = 10240

NC = 2
NS = 16
L = 16

COLS_PER_SC = NUM_COLS // NC
ROWS_PER_TILE = NUM_ROWS // NS
CHUNK = 64
N_MAIN = ROWS_PER_TILE // CHUNK
TAIL = ROWS_PER_TILE - N_MAIN * CHUNK
NBUF = 4
N_GROUPS = N_MAIN // NBUF
LOAD_AHEAD = 2
SEG_PER_TILE = N_SEG_PAD // NS
STRIP = CHUNK
N_STRIPS = SEG_PER_TILE // STRIP
LAST_SEGS = N_SEG - (NS - 1) * SEG_PER_TILE
LAST_FULL_STRIPS = LAST_SEGS // STRIP
LAST_PART = LAST_SEGS - LAST_FULL_STRIPS * STRIP


def _body(x_hbm, idx_hbm, out_hbm,
          idxb, rowsb, idxt_v, tail_v, ones_v, cnt_v,
          acc_sh, cntacc_sh, sems, out_sems):
    c = lax.axis_index("c")
    s = lax.axis_index("s")
    seg0 = s * SEG_PER_TILE
    row_base = s * ROWS_PER_TILE
    col0 = c * COLS_PER_SC

    zeros16 = jnp.zeros((L,), jnp.float32)
    ones16 = jnp.ones((L,), jnp.float32)

    def ones_body(i, _):
        ones_v[i, :] = ones16
        return 0

    lax.fori_loop(0, CHUNK, ones_body, 0)

    rg_base = s * (ROWS_PER_TILE // 8)
    RG_PER_CHUNK = CHUNK // 8

    def start_load(k, b):
        rg0 = rg_base + k * RG_PER_CHUNK
        for i in range(RG_PER_CHUNK):
            pltpu.async_copy(x_hbm.at[rg0 + i, c],
                             rowsb[b].at[pl.ds(i * 8, 8)], sems[b])
        pltpu.async_copy(idx_hbm.at[pl.ds(row_base + k * CHUNK, CHUNK)],
                         idxb[b], sems[b])

    def wait_load(b):
        for i in range(RG_PER_CHUNK):
            pltpu.make_async_copy(x_hbm.at[0, 0],
                                  rowsb[b].at[pl.ds(i * 8, 8)],
                                  sems[b]).wait()
        pltpu.make_async_copy(idx_hbm.at[pl.ds(row_base, CHUNK)],
                              idxb[b], sems[b]).wait()

    def start_scatter(k, b):
        pltpu.async_copy(rowsb[b], acc_sh.at[idxb[b]], sems[NBUF + b],
                         add=True)
        pltpu.async_copy(ones_v, cntacc_sh.at[idxb[b]], sems[NBUF + b],
                         add=True)

    def wait_scatter(b):
        pltpu.make_async_copy(rowsb[b], acc_sh.at[idxb[b]],
                              sems[NBUF + b]).wait()
        pltpu.make_async_copy(ones_v, cntacc_sh.at[idxb[b]],
                              sems[NBUF + b]).wait()

    def zero_body(i, _):
        for j in range(COLS_PER_SC // L):
            rowsb[0][i, pl.ds(j * L, L)] = zeros16
        cnt_v[i, :] = zeros16
        return 0

    lax.fori_loop(0, CHUNK, zero_body, 0)
    for st in range(N_STRIPS):
        pltpu.sync_copy(rowsb[0], acc_sh.at[pl.ds(seg0 + st * STRIP, STRIP)])
        pltpu.sync_copy(cnt_v, cntacc_sh.at[pl.ds(seg0 + st * STRIP, STRIP)])

    for b in range(LOAD_AHEAD):
        start_load(b, b)

    plsc.subcore_barrier()

    def slot(j, b, drain, load):
        wait_load(b)
        start_scatter(j, b)
        if drain:
            wait_scatter((b + LOAD_AHEAD) % NBUF)
        if load:
            start_load(j + LOAD_AHEAD, (b + LOAD_AHEAD) % NBUF)

    for b in range(NBUF):
        slot(b, b, b >= NBUF - LOAD_AHEAD, True)

    def group_body(g, _):
        j0 = g * NBUF
        for b in range(NBUF):
            slot(j0 + b, b, True, True)
        return 0

    lax.fori_loop(1, N_GROUPS - 1, group_body, 0)

    j0 = (N_GROUPS - 1) * NBUF
    for b in range(NBUF):
        slot(j0 + b, b, True, b < NBUF - LOAD_AHEAD)
    for b in range(LOAD_AHEAD, NBUF):
        wait_scatter(b)

    pltpu.sync_copy(idx_hbm.at[pl.ds(row_base + N_MAIN * CHUNK, TAIL)],
                    idxt_v)
    rg_tail = rg_base + N_MAIN * RG_PER_CHUNK
    for i in range(TAIL // 8):
        pltpu.sync_copy(x_hbm.at[rg_tail + i, c],
                        tail_v.at[pl.ds(i * 8, 8)])
    pltpu.sync_copy(tail_v, acc_sh.at[idxt_v], add=True)
    pltpu.sync_copy(ones_v.at[pl.ds(0, TAIL)], cntacc_sh.at[idxt_v], add=True)

    plsc.subcore_barrier()

    seg_rg0 = s * (SEG_PER_TILE // 8)

    def store_strip(st, b, n=STRIP):
        rg = seg_rg0 + st * (STRIP // 8)
        for i in range(n // 8):
            pltpu.async_copy(rowsb[b].at[pl.ds(i * 8, 8)],
                             out_hbm.at[rg + i, c], out_sems[b])

    def wait_strip(b, n=STRIP):
        for i in range(n // 8):
            pltpu.make_async_copy(rowsb[b].at[pl.ds(i * 8, 8)],
                                  out_hbm.at[0, 0], out_sems[b]).wait()

    def finalize_strip(st, b):
        pltpu.sync_copy(acc_sh.at[pl.ds(seg0 + st * STRIP, STRIP)], rowsb[b])
        pltpu.sync_copy(cntacc_sh.at[pl.ds(seg0 + st * STRIP, STRIP)], cnt_v)

        def div_body(i, _):
            inv = ones16 / jnp.maximum(cnt_v[i, :], ones16)
            for j in range(COLS_PER_SC // L):
                rowsb[b][i, pl.ds(j * L, L)] = (
                    rowsb[b][i, pl.ds(j * L, L)] * inv)
            return 0

        lax.fori_loop(0, STRIP, div_body, 0)

    def wait_out_full(st_done, b):
        wait_strip(b)

    def wait_out_part(st_done, b):
        wait_strip(b, LAST_PART)

    for st in range(N_STRIPS):
        b = st % 2
        if st >= 2:
            st_done = st - 2
            if st_done < LAST_FULL_STRIPS:
                wait_out_full(st_done, b)
            else:
                @pl.when(s < NS - 1)
                def _():
                    wait_out_full(st_done, b)

                if st_done == LAST_FULL_STRIPS:
                    @pl.when(s == NS - 1)
                    def _():
                        wait_out_part(st_done, b)

        finalize_strip(st, b)

        if st < LAST_FULL_STRIPS:
            store_strip(st, b)
        else:
            @pl.when(s < NS - 1)
            def _():
                store_strip(st, b)

            if st == LAST_FULL_STRIPS:
                @pl.when(s == NS - 1)
                def _():
                    store_strip(st, b, LAST_PART)

    @pl.when(s < NS - 1)
    def _():
        for st_done in (N_STRIPS - 2, N_STRIPS - 1):
            wait_out_full(st_done, st_done % 2)


def _mean_pool(x, index):
    run = pl.kernel(
        _body,
        out_type=jax.ShapeDtypeStruct((N_SEG // 8, NC, 8, COLS_PER_SC),
                                      jnp.float32),
        mesh=plsc.VectorSubcoreMesh(core_axis_name="c", subcore_axis_name="s"),
        compiler_params=pltpu.CompilerParams(use_tc_tiling_on_sc=False),
        scratch_types=[
            [pltpu.VMEM((CHUNK,), jnp.int32) for _ in range(NBUF)],
            [pltpu.VMEM((CHUNK, COLS_PER_SC), jnp.float32)
             for _ in range(NBUF)],
            pltpu.VMEM((TAIL,), jnp.int32),
            pltpu.VMEM((TAIL, COLS_PER_SC), jnp.float32),
            pltpu.VMEM((CHUNK, L), jnp.float32),
            pltpu.VMEM((STRIP, L), jnp.float32),
            pltpu.VMEM_SHARED((N_SEG_PAD, COLS_PER_SC), jnp.float32),
            pltpu.VMEM_SHARED((N_SEG_PAD, L), jnp.float32),
            [pltpu.SemaphoreType.DMA for _ in range(2 * NBUF)],
            [pltpu.SemaphoreType.DMA for _ in range(2)],
        ],
    )
    return run(x, index)


@jax.jit
def kernel(x, index):
    x4 = x.reshape(NUM_ROWS // 8, 8, NC, COLS_PER_SC).transpose(0, 2, 1, 3)
    out4 = _mean_pool(x4, index.astype(jnp.int32))
    return out4.transpose(0, 2, 1, 3).reshape(N_SEG, NUM_COLS)


# speedup vs baseline: 6.6479x; 1.0056x over previous
import jax
import jax.numpy as jnp
from jax import lax
from jax.experimental import pallas as pl
from jax.experimental.pallas import tpu as pltpu
from jax.experimental.pallas import tpu_sc as plsc

NUM_ROWS = 160000
NUM_COLS = 256
N_SEG = 10000
N_SEG_PAD = 10240

NC = 2
NS = 16
L = 16

COLS_PER_SC = NUM_COLS // NC
ROWS_PER_TILE = NUM_ROWS // NS
CHUNK = 64
N_MAIN = ROWS_PER_TILE // CHUNK
TAIL = ROWS_PER_TILE - N_MAIN * CHUNK
NBUF = 4
N_GROUPS = N_MAIN // NBUF
LOAD_AHEAD = 2
SEG_PER_TILE = N_SEG_PAD // NS
STRIP = CHUNK
N_STRIPS = SEG_PER_TILE // STRIP
LAST_SEGS = N_SEG - (NS - 1) * SEG_PER_TILE
LAST_FULL_STRIPS = LAST_SEGS // STRIP
LAST_PART = LAST_SEGS - LAST_FULL_STRIPS * STRIP


def _body(x_hbm, idx_hbm, dx_hbm, out_hbm,
          idxb, rowsb, idxt_v, tail_v, ones_v, cnt_v,
          acc_sh, cntacc_sh, sems, out_sems):
    c = lax.axis_index("c")
    s = lax.axis_index("s")
    seg0 = s * SEG_PER_TILE
    row_base = s * ROWS_PER_TILE
    col0 = c * COLS_PER_SC

    zeros16 = jnp.zeros((L,), jnp.float32)
    ones16 = jnp.ones((L,), jnp.float32)

    def ones_body(i, _):
        ones_v[i, :] = ones16
        return 0

    lax.fori_loop(0, CHUNK, ones_body, 0)

    rg_base = s * (ROWS_PER_TILE // 8)
    RG_PER_CHUNK = CHUNK // 8

    def start_load(k, b):
        rg0 = rg_base + k * RG_PER_CHUNK
        for i in range(RG_PER_CHUNK):
            pltpu.async_copy(x_hbm.at[rg0 + i, c],
                             rowsb[b].at[pl.ds(i * 8, 8)], sems[b])
        pltpu.async_copy(idx_hbm.at[pl.ds(row_base + k * CHUNK, CHUNK)],
                         idxb[b], sems[b])

    def wait_load(b):
        pltpu.make_async_copy(dx_hbm.at[pl.ds(0, CHUNK)], rowsb[b],
                              sems[b]).wait()
        pltpu.make_async_copy(idx_hbm.at[pl.ds(row_base, CHUNK)],
                              idxb[b], sems[b]).wait()

    def start_scatter(k, b):
        pltpu.async_copy(rowsb[b], acc_sh.at[idxb[b]], sems[NBUF + b],
                         add=True)
        pltpu.async_copy(ones_v, cntacc_sh.at[idxb[b]], sems[NBUF + b],
                         add=True)

    def wait_scatter(b):
        pltpu.make_async_copy(rowsb[b], acc_sh.at[idxb[b]],
                              sems[NBUF + b]).wait()
        pltpu.make_async_copy(ones_v, cntacc_sh.at[idxb[b]],
                              sems[NBUF + b]).wait()

    def zero_body(i, _):
        for j in range(COLS_PER_SC // L):
            rowsb[0][i, pl.ds(j * L, L)] = zeros16
        cnt_v[i, :] = zeros16
        return 0

    lax.fori_loop(0, CHUNK, zero_body, 0)
    for st in range(N_STRIPS):
        pltpu.sync_copy(rowsb[0], acc_sh.at[pl.ds(seg0 + st * STRIP, STRIP)])
        pltpu.sync_copy(cnt_v, cntacc_sh.at[pl.ds(seg0 + st * STRIP, STRIP)])

    for b in range(LOAD_AHEAD):
        start_load(b, b)

    plsc.subcore_barrier()

    def slot(j, b, drain, load):
        wait_load(b)
        start_scatter(j, b)
        if drain:
            wait_scatter((b + LOAD_AHEAD) % NBUF)
        if load:
            start_load(j + LOAD_AHEAD, (b + LOAD_AHEAD) % NBUF)

    for b in range(NBUF):
        slot(b, b, b >= NBUF - LOAD_AHEAD, True)

    def group_body(g, _):
        j0 = g * NBUF
        for b in range(NBUF):
            slot(j0 + b, b, True, True)
        return 0

    lax.fori_loop(1, N_GROUPS - 1, group_body, 0)

    j0 = (N_GROUPS - 1) * NBUF
    for b in range(NBUF):
        slot(j0 + b, b, True, b < NBUF - LOAD_AHEAD)
    for b in range(LOAD_AHEAD, NBUF):
        wait_scatter(b)

    pltpu.sync_copy(idx_hbm.at[pl.ds(row_base + N_MAIN * CHUNK, TAIL)],
                    idxt_v)
    rg_tail = rg_base + N_MAIN * RG_PER_CHUNK
    for i in range(TAIL // 8):
        pltpu.sync_copy(x_hbm.at[rg_tail + i, c],
                        tail_v.at[pl.ds(i * 8, 8)])
    pltpu.sync_copy(tail_v, acc_sh.at[idxt_v], add=True)
    pltpu.sync_copy(ones_v.at[pl.ds(0, TAIL)], cntacc_sh.at[idxt_v], add=True)

    plsc.subcore_barrier()

    seg_rg0 = s * (SEG_PER_TILE // 8)

    def store_strip(st, b, n=STRIP):
        rg = seg_rg0 + st * (STRIP // 8)
        for i in range(n // 8):
            pltpu.async_copy(rowsb[b].at[pl.ds(i * 8, 8)],
                             out_hbm.at[rg + i, c], out_sems[b])

    def wait_strip(b, n=STRIP):
        pltpu.make_async_copy(rowsb[b].at[pl.ds(0, n)],
                              dx_hbm.at[pl.ds(0, n)], out_sems[b]).wait()

    def finalize_strip(st, b):
        pltpu.sync_copy(acc_sh.at[pl.ds(seg0 + st * STRIP, STRIP)], rowsb[b])
        pltpu.sync_copy(cntacc_sh.at[pl.ds(seg0 + st * STRIP, STRIP)], cnt_v)

        def div_body(i, _):
            inv = ones16 / jnp.maximum(cnt_v[i, :], ones16)
            for j in range(COLS_PER_SC // L):
                rowsb[b][i, pl.ds(j * L, L)] = (
                    rowsb[b][i, pl.ds(j * L, L)] * inv)
            return 0

        lax.fori_loop(0, STRIP, div_body, 0)

    def wait_out_full(st_done, b):
        wait_strip(b)

    def wait_out_part(st_done, b):
        wait_strip(b, LAST_PART)

    for st in range(N_STRIPS):
        b = st % 2
        if st >= 2:
            st_done = st - 2
            if st_done < LAST_FULL_STRIPS:
                wait_out_full(st_done, b)
            else:
                @pl.when(s < NS - 1)
                def _():
                    wait_out_full(st_done, b)

                if st_done == LAST_FULL_STRIPS:
                    @pl.when(s == NS - 1)
                    def _():
                        wait_out_part(st_done, b)

        finalize_strip(st, b)

        if st < LAST_FULL_STRIPS:
            store_strip(st, b)
        else:
            @pl.when(s < NS - 1)
            def _():
                store_strip(st, b)

            if st == LAST_FULL_STRIPS:
                @pl.when(s == NS - 1)
                def _():
                    store_strip(st, b, LAST_PART)

    @pl.when(s < NS - 1)
    def _():
        for st_done in (N_STRIPS - 2, N_STRIPS - 1):
            wait_out_full(st_done, st_done % 2)


def _mean_pool(x, index, dx):
    run = pl.kernel(
        _body,
        out_type=jax.ShapeDtypeStruct((N_SEG // 8, NC, 8, COLS_PER_SC),
                                      jnp.float32),
        mesh=plsc.VectorSubcoreMesh(core_axis_name="c", subcore_axis_name="s"),
        compiler_params=pltpu.CompilerParams(use_tc_tiling_on_sc=False),
        scratch_types=[
            [pltpu.VMEM((CHUNK,), jnp.int32) for _ in range(NBUF)],
            [pltpu.VMEM((CHUNK, COLS_PER_SC), jnp.float32)
             for _ in range(NBUF)],
            pltpu.VMEM((TAIL,), jnp.int32),
            pltpu.VMEM((TAIL, COLS_PER_SC), jnp.float32),
            pltpu.VMEM((CHUNK, L), jnp.float32),
            pltpu.VMEM((STRIP, L), jnp.float32),
            pltpu.VMEM_SHARED((N_SEG_PAD, COLS_PER_SC), jnp.float32),
            pltpu.VMEM_SHARED((N_SEG_PAD, L), jnp.float32),
            [pltpu.SemaphoreType.DMA for _ in range(2 * NBUF)],
            [pltpu.SemaphoreType.DMA for _ in range(2)],
        ],
    )
    return run(x, index, dx)


@jax.jit
def kernel(x, index):
    x4 = x.reshape(NUM_ROWS // 8, 8, NC, COLS_PER_SC).transpose(0, 2, 1, 3)
    dx = jnp.zeros((CHUNK, COLS_PER_SC), jnp.float32)
    out4 = _mean_pool(x4, index.astype(jnp.int32), dx)
    return out4.transpose(0, 2, 1, 3).reshape(N_SEG, NUM_COLS)


# speedup vs baseline: 6.7448x; 1.0146x over previous
import jax
import jax.numpy as jnp
from jax import lax
from jax.experimental import pallas as pl
from jax.experimental.pallas import tpu as pltpu
from jax.experimental.pallas import tpu_sc as plsc

NUM_ROWS = 160000
NUM_COLS = 256
N_SEG = 10000
N_SEG_PAD = 10240

NC = 2
NS = 16
L = 16

COLS_PER_SC = NUM_COLS // NC
ROWS_PER_TILE = NUM_ROWS // NS
CHUNK = 64
N_MAIN = ROWS_PER_TILE // CHUNK
TAIL = ROWS_PER_TILE - N_MAIN * CHUNK
NBUF = 4
N_GROUPS = N_MAIN // NBUF
LOAD_AHEAD = 2
SEG_PER_TILE = N_SEG_PAD // NS
STRIP = CHUNK
N_STRIPS = SEG_PER_TILE // STRIP
LAST_SEGS = N_SEG - (NS - 1) * SEG_PER_TILE
LAST_FULL_STRIPS = LAST_SEGS // STRIP
LAST_PART = LAST_SEGS - LAST_FULL_STRIPS * STRIP


def _body(x_hbm, idx_hbm, dx_hbm, out_hbm,
          idxb, rowsb, idxt_v, tail_v, ones_v, cnt_v,
          acc_sh, cntacc_sh, sems, out_sems):
    c = lax.axis_index("c")
    s = lax.axis_index("s")
    seg0 = s * SEG_PER_TILE
    row_base = s * ROWS_PER_TILE
    col0 = c * COLS_PER_SC

    zeros16 = jnp.zeros((L,), jnp.float32)
    ones16 = jnp.ones((L,), jnp.float32)

    def ones_body(i, _):
        ones_v[i, :] = ones16
        return 0

    lax.fori_loop(0, CHUNK, ones_body, 0)

    rg_base = s * (ROWS_PER_TILE // 8)
    RG_PER_CHUNK = CHUNK // 8

    def start_load(k, b):
        rg0 = rg_base + k * RG_PER_CHUNK
        for i in range(RG_PER_CHUNK):
            pltpu.async_copy(x_hbm.at[rg0 + i, c],
                             rowsb[b].at[pl.ds(i * 8, 8)], sems[b])
        pltpu.async_copy(idx_hbm.at[pl.ds(row_base + k * CHUNK, CHUNK)],
                         idxb[b], sems[b])

    def wait_load(b):
        pltpu.make_async_copy(dx_hbm.at[pl.ds(0, CHUNK)], rowsb[b],
                              sems[b]).wait()
        pltpu.make_async_copy(idx_hbm.at[pl.ds(row_base, CHUNK)],
                              idxb[b], sems[b]).wait()

    def start_scatter(k, b):
        pltpu.async_copy(rowsb[b], acc_sh.at[idxb[b]], sems[NBUF + b],
                         add=True)
        pltpu.async_copy(ones_v, cntacc_sh.at[idxb[b]], sems[NBUF + b],
                         add=True)

    def wait_scatter(b):
        pltpu.make_async_copy(rowsb[b], acc_sh.at[idxb[b]],
                              sems[NBUF + b]).wait()
        pltpu.make_async_copy(ones_v, cntacc_sh.at[idxb[b]],
                              sems[NBUF + b]).wait()

    for b in range(LOAD_AHEAD):
        start_load(b, b)

    def zero_body(i, _):
        for j in range(COLS_PER_SC // L):
            rowsb[NBUF - 1][i, pl.ds(j * L, L)] = zeros16
        cnt_v[i, :] = zeros16
        return 0

    lax.fori_loop(0, CHUNK, zero_body, 0)
    for st in range(N_STRIPS):
        pltpu.async_copy(rowsb[NBUF - 1],
                         acc_sh.at[pl.ds(seg0 + st * STRIP, STRIP)],
                         out_sems[0])
        pltpu.async_copy(cnt_v, cntacc_sh.at[pl.ds(seg0 + st * STRIP, STRIP)],
                         out_sems[1])
    for st in range(N_STRIPS):
        pltpu.make_async_copy(dx_hbm.at[pl.ds(0, CHUNK)], rowsb[NBUF - 1],
                              out_sems[0]).wait()
        pltpu.make_async_copy(dx_hbm.at[pl.ds(0, 8)],
                              rowsb[NBUF - 1].at[pl.ds(0, 8)],
                              out_sems[1]).wait()

    plsc.subcore_barrier()

    def slot(j, b, drain, load):
        wait_load(b)
        start_scatter(j, b)
        if drain:
            wait_scatter((b + LOAD_AHEAD) % NBUF)
        if load:
            start_load(j + LOAD_AHEAD, (b + LOAD_AHEAD) % NBUF)

    for b in range(NBUF):
        slot(b, b, b >= NBUF - LOAD_AHEAD, True)

    def group_body(g, _):
        j0 = g * NBUF
        for b in range(NBUF):
            slot(j0 + b, b, True, True)
        return 0

    lax.fori_loop(1, N_GROUPS - 1, group_body, 0)

    j0 = (N_GROUPS - 1) * NBUF
    for b in range(NBUF):
        slot(j0 + b, b, True, b < NBUF - LOAD_AHEAD)
    for b in range(LOAD_AHEAD, NBUF):
        wait_scatter(b)

    pltpu.sync_copy(idx_hbm.at[pl.ds(row_base + N_MAIN * CHUNK, TAIL)],
                    idxt_v)
    rg_tail = rg_base + N_MAIN * RG_PER_CHUNK
    for i in range(TAIL // 8):
        pltpu.sync_copy(x_hbm.at[rg_tail + i, c],
                        tail_v.at[pl.ds(i * 8, 8)])
    pltpu.sync_copy(tail_v, acc_sh.at[idxt_v], add=True)
    pltpu.sync_copy(ones_v.at[pl.ds(0, TAIL)], cntacc_sh.at[idxt_v], add=True)

    plsc.subcore_barrier()

    seg_rg0 = s * (SEG_PER_TILE // 8)

    def store_strip(st, b, n=STRIP):
        rg = seg_rg0 + st * (STRIP // 8)
        for i in range(n // 8):
            pltpu.async_copy(rowsb[b].at[pl.ds(i * 8, 8)],
                             out_hbm.at[rg + i, c], out_sems[b])

    def wait_strip(b, n=STRIP):
        pltpu.make_async_copy(rowsb[b].at[pl.ds(0, n)],
                              dx_hbm.at[pl.ds(0, n)], out_sems[b]).wait()

    def finalize_strip(st, b):
        pltpu.sync_copy(acc_sh.at[pl.ds(seg0 + st * STRIP, STRIP)], rowsb[b])
        pltpu.sync_copy(cntacc_sh.at[pl.ds(seg0 + st * STRIP, STRIP)], cnt_v)

        def div_body(i, _):
            inv = ones16 / jnp.maximum(cnt_v[i, :], ones16)
            for j in range(COLS_PER_SC // L):
                rowsb[b][i, pl.ds(j * L, L)] = (
                    rowsb[b][i, pl.ds(j * L, L)] * inv)
            return 0

        lax.fori_loop(0, STRIP, div_body, 0)

    def wait_out_full(st_done, b):
        wait_strip(b)

    def wait_out_part(st_done, b):
        wait_strip(b, LAST_PART)

    for st in range(N_STRIPS):
        b = st % 2
        if st >= 2:
            st_done = st - 2
            if st_done < LAST_FULL_STRIPS:
                wait_out_full(st_done, b)
            else:
                @pl.when(s < NS - 1)
                def _():
                    wait_out_full(st_done, b)

                if st_done == LAST_FULL_STRIPS:
                    @pl.when(s == NS - 1)
                    def _():
                        wait_out_part(st_done, b)

        finalize_strip(st, b)

        if st < LAST_FULL_STRIPS:
            store_strip(st, b)
        else:
            @pl.when(s < NS - 1)
            def _():
                store_strip(st, b)

            if st == LAST_FULL_STRIPS:
                @pl.when(s == NS - 1)
                def _():
                    store_strip(st, b, LAST_PART)

    @pl.when(s < NS - 1)
    def _():
        for st_done in (N_STRIPS - 2, N_STRIPS - 1):
            wait_out_full(st_done, st_done % 2)


def _mean_pool(x, index, dx):
    run = pl.kernel(
        _body,
        out_type=jax.ShapeDtypeStruct((N_SEG // 8, NC, 8, COLS_PER_SC),
                                      jnp.float32),
        mesh=plsc.VectorSubcoreMesh(core_axis_name="c", subcore_axis_name="s"),
        compiler_params=pltpu.CompilerParams(use_tc_tiling_on_sc=False),
        scratch_types=[
            [pltpu.VMEM((CHUNK,), jnp.int32) for _ in range(NBUF)],
            [pltpu.VMEM((CHUNK, COLS_PER_SC), jnp.float32)
             for _ in range(NBUF)],
            pltpu.VMEM((TAIL,), jnp.int32),
            pltpu.VMEM((TAIL, COLS_PER_SC), jnp.float32),
            pltpu.VMEM((CHUNK, L), jnp.float32),
            pltpu.VMEM((STRIP, L), jnp.float32),
            pltpu.VMEM_SHARED((N_SEG_PAD, COLS_PER_SC), jnp.float32),
            pltpu.VMEM_SHARED((N_SEG_PAD, L), jnp.float32),
            [pltpu.SemaphoreType.DMA for _ in range(2 * NBUF)],
            [pltpu.SemaphoreType.DMA for _ in range(2)],
        ],
    )
    return run(x, index, dx)


@jax.jit
def kernel(x, index):
    x4 = x.reshape(NUM_ROWS // 8, 8, NC, COLS_PER_SC).transpose(0, 2, 1, 3)
    dx = jnp.zeros((CHUNK, COLS_PER_SC), jnp.float32)
    out4 = _mean_pool(x4, index.astype(jnp.int32), dx)
    return out4.transpose(0, 2, 1, 3).reshape(N_SEG, NUM_COLS)


# speedup vs baseline: 7.2821x; 1.0797x over previous
import jax
import jax.numpy as jnp
from jax import lax
from jax.experimental import pallas as pl
from jax.experimental.pallas import tpu as pltpu
from jax.experimental.pallas import tpu_sc as plsc

NUM_ROWS = 160000
NUM_COLS = 256
N_SEG = 10000
N_SEG_PAD = 10240

NC = 2
NS = 16
L = 16

COLS_PER_SC = NUM_COLS // NC
ROWS_PER_TILE = NUM_ROWS // NS
CHUNK = 64
N_MAIN = ROWS_PER_TILE // CHUNK
TAIL = ROWS_PER_TILE - N_MAIN * CHUNK
NBUF = 4
N_GROUPS = N_MAIN // NBUF
LOAD_AHEAD = 3
SEG_PER_TILE = N_SEG_PAD // NS
STRIP = CHUNK
N_STRIPS = SEG_PER_TILE // STRIP
LAST_SEGS = N_SEG - (NS - 1) * SEG_PER_TILE
LAST_FULL_STRIPS = LAST_SEGS // STRIP
LAST_PART = LAST_SEGS - LAST_FULL_STRIPS * STRIP


def _body(x_hbm, idx_hbm, dx_hbm, out_hbm,
          idxb, rowsb, idxt_v, tail_v, ones_v, cnt_v,
          acc_sh, cntacc_sh, sems, out_sems):
    c = lax.axis_index("c")
    s = lax.axis_index("s")
    seg0 = s * SEG_PER_TILE
    row_base = s * ROWS_PER_TILE
    col0 = c * COLS_PER_SC

    zeros16 = jnp.zeros((L,), jnp.float32)
    ones16 = jnp.ones((L,), jnp.float32)

    def ones_body(i, _):
        ones_v[i, :] = ones16
        return 0

    lax.fori_loop(0, CHUNK, ones_body, 0)

    rg_base = s * (ROWS_PER_TILE // 8)
    RG_PER_CHUNK = CHUNK // 8

    def start_load(k, b):
        rg0 = rg_base + k * RG_PER_CHUNK
        for i in range(RG_PER_CHUNK):
            pltpu.async_copy(x_hbm.at[rg0 + i, c],
                             rowsb[b].at[pl.ds(i * 8, 8)], sems[b])
        pltpu.async_copy(idx_hbm.at[pl.ds(row_base + k * CHUNK, CHUNK)],
                         idxb[b], sems[b])

    def wait_load(b):
        pltpu.make_async_copy(dx_hbm.at[pl.ds(0, CHUNK)], rowsb[b],
                              sems[b]).wait()
        pltpu.make_async_copy(idx_hbm.at[pl.ds(row_base, CHUNK)],
                              idxb[b], sems[b]).wait()

    def start_scatter(k, b):
        pltpu.async_copy(rowsb[b], acc_sh.at[idxb[b]], sems[NBUF + b],
                         add=True)
        pltpu.async_copy(ones_v, cntacc_sh.at[idxb[b]], sems[NBUF + b],
                         add=True)

    def wait_scatter(b):
        pltpu.make_async_copy(rowsb[b], acc_sh.at[idxb[b]],
                              sems[NBUF + b]).wait()
        pltpu.make_async_copy(ones_v, cntacc_sh.at[idxb[b]],
                              sems[NBUF + b]).wait()

    for b in range(LOAD_AHEAD):
        start_load(b, b)

    def zero_body(i, _):
        for j in range(COLS_PER_SC // L):
            rowsb[NBUF - 1][i, pl.ds(j * L, L)] = zeros16
        cnt_v[i, :] = zeros16
        return 0

    lax.fori_loop(0, CHUNK, zero_body, 0)
    for st in range(N_STRIPS):
        pltpu.async_copy(rowsb[NBUF - 1],
                         acc_sh.at[pl.ds(seg0 + st * STRIP, STRIP)],
                         out_sems[0])
        pltpu.async_copy(cnt_v, cntacc_sh.at[pl.ds(seg0 + st * STRIP, STRIP)],
                         out_sems[1])
    for st in range(N_STRIPS):
        pltpu.make_async_copy(dx_hbm.at[pl.ds(0, CHUNK)], rowsb[NBUF - 1],
                              out_sems[0]).wait()
        pltpu.make_async_copy(dx_hbm.at[pl.ds(0, 8)],
                              rowsb[NBUF - 1].at[pl.ds(0, 8)],
                              out_sems[1]).wait()

    plsc.subcore_barrier()

    def slot(j, b, drain, load):
        wait_load(b)
        start_scatter(j, b)
        if drain:
            wait_scatter((b + LOAD_AHEAD) % NBUF)
        if load:
            start_load(j + LOAD_AHEAD, (b + LOAD_AHEAD) % NBUF)

    for b in range(NBUF):
        slot(b, b, b >= NBUF - LOAD_AHEAD, True)

    def group_body(g, _):
        j0 = g * NBUF
        for b in range(NBUF):
            slot(j0 + b, b, True, True)
        return 0

    lax.fori_loop(1, N_GROUPS - 1, group_body, 0)

    j0 = (N_GROUPS - 1) * NBUF
    for b in range(NBUF):
        slot(j0 + b, b, True, b < NBUF - LOAD_AHEAD)
    for b in range(LOAD_AHEAD, NBUF):
        wait_scatter(b)

    pltpu.sync_copy(idx_hbm.at[pl.ds(row_base + N_MAIN * CHUNK, TAIL)],
                    idxt_v)
    rg_tail = rg_base + N_MAIN * RG_PER_CHUNK
    for i in range(TAIL // 8):
        pltpu.sync_copy(x_hbm.at[rg_tail + i, c],
                        tail_v.at[pl.ds(i * 8, 8)])
    pltpu.sync_copy(tail_v, acc_sh.at[idxt_v], add=True)
    pltpu.sync_copy(ones_v.at[pl.ds(0, TAIL)], cntacc_sh.at[idxt_v], add=True)

    plsc.subcore_barrier()

    seg_rg0 = s * (SEG_PER_TILE // 8)

    def store_strip(st, b, n=STRIP):
        rg = seg_rg0 + st * (STRIP // 8)
        for i in range(n // 8):
            pltpu.async_copy(rowsb[b].at[pl.ds(i * 8, 8)],
                             out_hbm.at[rg + i, c], out_sems[b])

    def wait_strip(b, n=STRIP):
        pltpu.make_async_copy(rowsb[b].at[pl.ds(0, n)],
                              dx_hbm.at[pl.ds(0, n)], out_sems[b]).wait()

    def finalize_strip(st, b):
        pltpu.sync_copy(acc_sh.at[pl.ds(seg0 + st * STRIP, STRIP)], rowsb[b])
        pltpu.sync_copy(cntacc_sh.at[pl.ds(seg0 + st * STRIP, STRIP)], cnt_v)

        def div_body(i, _):
            inv = ones16 / jnp.maximum(cnt_v[i, :], ones16)
            for j in range(COLS_PER_SC // L):
                rowsb[b][i, pl.ds(j * L, L)] = (
                    rowsb[b][i, pl.ds(j * L, L)] * inv)
            return 0

        lax.fori_loop(0, STRIP, div_body, 0)

    def wait_out_full(st_done, b):
        wait_strip(b)

    def wait_out_part(st_done, b):
        wait_strip(b, LAST_PART)

    for st in range(N_STRIPS):
        b = st % 2
        if st >= 2:
            st_done = st - 2
            if st_done < LAST_FULL_STRIPS:
                wait_out_full(st_done, b)
            else:
                @pl.when(s < NS - 1)
                def _():
                    wait_out_full(st_done, b)

                if st_done == LAST_FULL_STRIPS:
                    @pl.when(s == NS - 1)
                    def _():
                        wait_out_part(st_done, b)

        finalize_strip(st, b)

        if st < LAST_FULL_STRIPS:
            store_strip(st, b)
        else:
            @pl.when(s < NS - 1)
            def _():
                store_strip(st, b)

            if st == LAST_FULL_STRIPS:
                @pl.when(s == NS - 1)
                def _():
                    store_strip(st, b, LAST_PART)

    @pl.when(s < NS - 1)
    def _():
        for st_done in (N_STRIPS - 2, N_STRIPS - 1):
            wait_out_full(st_done, st_done % 2)


def _mean_pool(x, index, dx):
    run = pl.kernel(
        _body,
        out_type=jax.ShapeDtypeStruct((N_SEG // 8, NC, 8, COLS_PER_SC),
                                      jnp.float32),
        mesh=plsc.VectorSubcoreMesh(core_axis_name="c", subcore_axis_name="s"),
        compiler_params=pltpu.CompilerParams(use_tc_tiling_on_sc=False),
        scratch_types=[
            [pltpu.VMEM((CHUNK,), jnp.int32) for _ in range(NBUF)],
            [pltpu.VMEM((CHUNK, COLS_PER_SC), jnp.float32)
             for _ in range(NBUF)],
            pltpu.VMEM((TAIL,), jnp.int32),
            pltpu.VMEM((TAIL, COLS_PER_SC), jnp.float32),
            pltpu.VMEM((CHUNK, L), jnp.float32),
            pltpu.VMEM((STRIP, L), jnp.float32),
            pltpu.VMEM_SHARED((N_SEG_PAD, COLS_PER_SC), jnp.float32),
            pltpu.VMEM_SHARED((N_SEG_PAD, L), jnp.float32),
            [pltpu.SemaphoreType.DMA for _ in range(2 * NBUF)],
            [pltpu.SemaphoreType.DMA for _ in range(2)],
        ],
    )
    return run(x, index, dx)


@jax.jit
def kernel(x, index):
    x4 = x.reshape(NUM_ROWS // 8, 8, NC, COLS_PER_SC).transpose(0, 2, 1, 3)
    dx = jnp.zeros((CHUNK, COLS_PER_SC), jnp.float32)
    out4 = _mean_pool(x4, index.astype(jnp.int32), dx)
    return out4.transpose(0, 2, 1, 3).reshape(N_SEG, NUM_COLS)


# speedup vs baseline: 7.4001x; 1.0162x over previous
import jax
import jax.numpy as jnp
from jax import lax
from jax.experimental import pallas as pl
from jax.experimental.pallas import tpu as pltpu
from jax.experimental.pallas import tpu_sc as plsc

NUM_ROWS = 160000
NUM_COLS = 256
N_SEG = 10000
N_SEG_PAD = 10240

NC = 2
NS = 16
L = 16

COLS_PER_SC = NUM_COLS // NC
ROWS_PER_TILE = NUM_ROWS // NS
CHUNK = 64
N_MAIN = ROWS_PER_TILE // CHUNK
TAIL = ROWS_PER_TILE - N_MAIN * CHUNK
NBUF = 4
N_GROUPS = N_MAIN // NBUF
LOAD_AHEAD = 3
SEG_PER_TILE = N_SEG_PAD // NS
STRIP = CHUNK
N_STRIPS = SEG_PER_TILE // STRIP
LAST_SEGS = N_SEG - (NS - 1) * SEG_PER_TILE
LAST_FULL_STRIPS = LAST_SEGS // STRIP
LAST_PART = LAST_SEGS - LAST_FULL_STRIPS * STRIP


def _body(x_hbm, idx_hbm, dx_hbm, out_hbm,
          idxb, rowsb, idxt_v, tail_v, ones_v, cnt_v,
          acc_sh, cntacc_sh, sems, out_sems):
    c = lax.axis_index("c")
    s = lax.axis_index("s")
    seg0 = s * SEG_PER_TILE
    row_base = s * ROWS_PER_TILE
    col0 = c * COLS_PER_SC

    zeros16 = jnp.zeros((L,), jnp.float32)
    ones16 = jnp.ones((L,), jnp.float32)

    def ones_body(i, _):
        ones_v[i, :] = ones16
        return 0

    lax.fori_loop(0, CHUNK, ones_body, 0)

    rg_base = s * (ROWS_PER_TILE // 8)
    RG_PER_CHUNK = CHUNK // 8

    def start_load(k, b):
        rg0 = rg_base + k * RG_PER_CHUNK
        for i in range(RG_PER_CHUNK):
            pltpu.async_copy(x_hbm.at[rg0 + i, c],
                             rowsb[b].at[pl.ds(i * 8, 8)], sems[b])
        pltpu.async_copy(idx_hbm.at[pl.ds(row_base + k * CHUNK, CHUNK)],
                         idxb[b], sems[b])

    def wait_load(b):
        pltpu.make_async_copy(dx_hbm.at[pl.ds(0, CHUNK)], rowsb[b],
                              sems[b]).wait()
        pltpu.make_async_copy(idx_hbm.at[pl.ds(row_base, CHUNK)],
                              idxb[b], sems[b]).wait()

    def start_scatter(k, b):
        pltpu.async_copy(rowsb[b], acc_sh.at[idxb[b]], sems[NBUF + b],
                         add=True)
        pltpu.async_copy(ones_v, cntacc_sh.at[idxb[b]], sems[NBUF + b],
                         add=True)

    def wait_scatter(b):
        pltpu.make_async_copy(rowsb[b], acc_sh.at[idxb[b]],
                              sems[NBUF + b]).wait()
        pltpu.make_async_copy(ones_v, cntacc_sh.at[idxb[b]],
                              sems[NBUF + b]).wait()

    for b in range(LOAD_AHEAD):
        start_load(b, b)

    def zero_body(i, _):
        for j in range(COLS_PER_SC // L):
            rowsb[NBUF - 1][i, pl.ds(j * L, L)] = zeros16
        cnt_v[i, :] = zeros16
        return 0

    lax.fori_loop(0, CHUNK, zero_body, 0)
    for st in range(N_STRIPS):
        pltpu.async_copy(rowsb[NBUF - 1],
                         acc_sh.at[pl.ds(seg0 + st * STRIP, STRIP)],
                         out_sems[0])
        pltpu.async_copy(cnt_v, cntacc_sh.at[pl.ds(seg0 + st * STRIP, STRIP)],
                         out_sems[1])
    for st in range(N_STRIPS):
        pltpu.make_async_copy(dx_hbm.at[pl.ds(0, CHUNK)], rowsb[NBUF - 1],
                              out_sems[0]).wait()
        pltpu.make_async_copy(dx_hbm.at[pl.ds(0, 8)],
                              rowsb[NBUF - 1].at[pl.ds(0, 8)],
                              out_sems[1]).wait()

    plsc.subcore_barrier()

    def slot(j, b, drain, load):
        wait_load(b)
        if drain:
            wait_scatter((b + LOAD_AHEAD) % NBUF)
        if load:
            start_load(j + LOAD_AHEAD, (b + LOAD_AHEAD) % NBUF)
        start_scatter(j, b)

    for b in range(NBUF):
        slot(b, b, b >= NBUF - LOAD_AHEAD, True)

    def group_body(g, _):
        j0 = g * NBUF
        for b in range(NBUF):
            slot(j0 + b, b, True, True)
        return 0

    lax.fori_loop(1, N_GROUPS - 1, group_body, 0)

    j0 = (N_GROUPS - 1) * NBUF
    for b in range(NBUF):
        slot(j0 + b, b, True, b < NBUF - LOAD_AHEAD)
    for b in range(LOAD_AHEAD, NBUF):
        wait_scatter(b)

    pltpu.sync_copy(idx_hbm.at[pl.ds(row_base + N_MAIN * CHUNK, TAIL)],
                    idxt_v)
    rg_tail = rg_base + N_MAIN * RG_PER_CHUNK
    for i in range(TAIL // 8):
        pltpu.sync_copy(x_hbm.at[rg_tail + i, c],
                        tail_v.at[pl.ds(i * 8, 8)])
    pltpu.sync_copy(tail_v, acc_sh.at[idxt_v], add=True)
    pltpu.sync_copy(ones_v.at[pl.ds(0, TAIL)], cntacc_sh.at[idxt_v], add=True)

    plsc.subcore_barrier()

    seg_rg0 = s * (SEG_PER_TILE // 8)

    def store_strip(st, b, n=STRIP):
        rg = seg_rg0 + st * (STRIP // 8)
        for i in range(n // 8):
            pltpu.async_copy(rowsb[b].at[pl.ds(i * 8, 8)],
                             out_hbm.at[rg + i, c], out_sems[b])

    def wait_strip(b, n=STRIP):
        pltpu.make_async_copy(rowsb[b].at[pl.ds(0, n)],
                              dx_hbm.at[pl.ds(0, n)], out_sems[b]).wait()

    def finalize_strip(st, b):
        pltpu.sync_copy(acc_sh.at[pl.ds(seg0 + st * STRIP, STRIP)], rowsb[b])
        pltpu.sync_copy(cntacc_sh.at[pl.ds(seg0 + st * STRIP, STRIP)], cnt_v)

        def div_body(i, _):
            inv = ones16 / jnp.maximum(cnt_v[i, :], ones16)
            for j in range(COLS_PER_SC // L):
                rowsb[b][i, pl.ds(j * L, L)] = (
                    rowsb[b][i, pl.ds(j * L, L)] * inv)
            return 0

        lax.fori_loop(0, STRIP, div_body, 0)

    def wait_out_full(st_done, b):
        wait_strip(b)

    def wait_out_part(st_done, b):
        wait_strip(b, LAST_PART)

    for st in range(N_STRIPS):
        b = st % 2
        if st >= 2:
            st_done = st - 2
            if st_done < LAST_FULL_STRIPS:
                wait_out_full(st_done, b)
            else:
                @pl.when(s < NS - 1)
                def _():
                    wait_out_full(st_done, b)

                if st_done == LAST_FULL_STRIPS:
                    @pl.when(s == NS - 1)
                    def _():
                        wait_out_part(st_done, b)

        finalize_strip(st, b)

        if st < LAST_FULL_STRIPS:
            store_strip(st, b)
        else:
            @pl.when(s < NS - 1)
            def _():
                store_strip(st, b)

            if st == LAST_FULL_STRIPS:
                @pl.when(s == NS - 1)
                def _():
                    store_strip(st, b, LAST_PART)

    @pl.when(s < NS - 1)
    def _():
        for st_done in (N_STRIPS - 2, N_STRIPS - 1):
            wait_out_full(st_done, st_done % 2)


def _mean_pool(x, index, dx):
    run = pl.kernel(
        _body,
        out_type=jax.ShapeDtypeStruct((N_SEG // 8, NC, 8, COLS_PER_SC),
                                      jnp.float32),
        mesh=plsc.VectorSubcoreMesh(core_axis_name="c", subcore_axis_name="s"),
        compiler_params=pltpu.CompilerParams(use_tc_tiling_on_sc=False),
        scratch_types=[
            [pltpu.VMEM((CHUNK,), jnp.int32) for _ in range(NBUF)],
            [pltpu.VMEM((CHUNK, COLS_PER_SC), jnp.float32)
             for _ in range(NBUF)],
            pltpu.VMEM((TAIL,), jnp.int32),
            pltpu.VMEM((TAIL, COLS_PER_SC), jnp.float32),
            pltpu.VMEM((CHUNK, L), jnp.float32),
            pltpu.VMEM((STRIP, L), jnp.float32),
            pltpu.VMEM_SHARED((N_SEG_PAD, COLS_PER_SC), jnp.float32),
            pltpu.VMEM_SHARED((N_SEG_PAD, L), jnp.float32),
            [pltpu.SemaphoreType.DMA for _ in range(2 * NBUF)],
            [pltpu.SemaphoreType.DMA for _ in range(2)],
        ],
    )
    return run(x, index, dx)


@jax.jit
def kernel(x, index):
    x4 = x.reshape(NUM_ROWS // 8, 8, NC, COLS_PER_SC).transpose(0, 2, 1, 3)
    dx = jnp.zeros((CHUNK, COLS_PER_SC), jnp.float32)
    out4 = _mean_pool(x4, index.astype(jnp.int32), dx)
    return out4.transpose(0, 2, 1, 3).reshape(N_SEG, NUM_COLS)


# speedup vs baseline: 7.5315x; 1.0177x over previous
import jax
import jax.numpy as jnp
from jax import lax
from jax.experimental import pallas as pl
from jax.experimental.pallas import tpu as pltpu
from jax.experimental.pallas import tpu_sc as plsc

NUM_ROWS = 160000
NUM_COLS = 256
N_SEG = 10000
N_SEG_PAD = 10240

NC = 2
NS = 16
L = 16

COLS_PER_SC = NUM_COLS // NC
ROWS_PER_TILE = NUM_ROWS // NS
CHUNK = 64
N_MAIN = ROWS_PER_TILE // CHUNK
TAIL = ROWS_PER_TILE - N_MAIN * CHUNK
NBUF = 4
N_GROUPS = N_MAIN // NBUF
LOAD_AHEAD = 3
SEG_PER_TILE = N_SEG_PAD // NS
STRIP = CHUNK
N_STRIPS = SEG_PER_TILE // STRIP
LAST_SEGS = N_SEG - (NS - 1) * SEG_PER_TILE
LAST_FULL_STRIPS = LAST_SEGS // STRIP
LAST_PART = LAST_SEGS - LAST_FULL_STRIPS * STRIP


def _body(x_hbm, idx_hbm, dx_hbm, out_hbm,
          idxb, rowsb, idxt_v, tail_v, ones_v, cnt_v, cntb1,
          acc_sh, cntacc_sh, sems, out_sems, rsems):
    c = lax.axis_index("c")
    s = lax.axis_index("s")
    seg0 = s * SEG_PER_TILE
    row_base = s * ROWS_PER_TILE
    col0 = c * COLS_PER_SC

    zeros16 = jnp.zeros((L,), jnp.float32)
    ones16 = jnp.ones((L,), jnp.float32)

    def ones_body(i, _):
        ones_v[i, :] = ones16
        return 0

    lax.fori_loop(0, CHUNK, ones_body, 0)

    rg_base = s * (ROWS_PER_TILE // 8)
    RG_PER_CHUNK = CHUNK // 8

    def start_load(k, b):
        rg0 = rg_base + k * RG_PER_CHUNK
        for i in range(RG_PER_CHUNK):
            pltpu.async_copy(x_hbm.at[rg0 + i, c],
                             rowsb[b].at[pl.ds(i * 8, 8)], sems[b])
        pltpu.async_copy(idx_hbm.at[pl.ds(row_base + k * CHUNK, CHUNK)],
                         idxb[b], sems[b])

    def wait_load(b):
        pltpu.make_async_copy(dx_hbm.at[pl.ds(0, CHUNK)], rowsb[b],
                              sems[b]).wait()
        pltpu.make_async_copy(idx_hbm.at[pl.ds(row_base, CHUNK)],
                              idxb[b], sems[b]).wait()

    def start_scatter(k, b):
        pltpu.async_copy(rowsb[b], acc_sh.at[idxb[b]], sems[NBUF + b],
                         add=True)
        pltpu.async_copy(ones_v, cntacc_sh.at[idxb[b]], sems[NBUF + b],
                         add=True)

    def wait_scatter(b):
        pltpu.make_async_copy(rowsb[b], acc_sh.at[idxb[b]],
                              sems[NBUF + b]).wait()
        pltpu.make_async_copy(ones_v, cntacc_sh.at[idxb[b]],
                              sems[NBUF + b]).wait()

    for b in range(LOAD_AHEAD):
        start_load(b, b)

    def zero_body(i, _):
        for j in range(COLS_PER_SC // L):
            rowsb[NBUF - 1][i, pl.ds(j * L, L)] = zeros16
        cnt_v[i, :] = zeros16
        return 0

    lax.fori_loop(0, CHUNK, zero_body, 0)
    for st in range(N_STRIPS):
        pltpu.async_copy(rowsb[NBUF - 1],
                         acc_sh.at[pl.ds(seg0 + st * STRIP, STRIP)],
                         out_sems[0])
        pltpu.async_copy(cnt_v, cntacc_sh.at[pl.ds(seg0 + st * STRIP, STRIP)],
                         out_sems[1])
    for st in range(N_STRIPS):
        pltpu.make_async_copy(dx_hbm.at[pl.ds(0, CHUNK)], rowsb[NBUF - 1],
                              out_sems[0]).wait()
        pltpu.make_async_copy(dx_hbm.at[pl.ds(0, 8)],
                              rowsb[NBUF - 1].at[pl.ds(0, 8)],
                              out_sems[1]).wait()

    plsc.subcore_barrier()

    def slot(j, b, drain, load):
        wait_load(b)
        if drain:
            wait_scatter((b + LOAD_AHEAD) % NBUF)
        if load:
            start_load(j + LOAD_AHEAD, (b + LOAD_AHEAD) % NBUF)
        start_scatter(j, b)

    for b in range(NBUF):
        slot(b, b, b >= NBUF - LOAD_AHEAD, True)

    def group_body(g, _):
        j0 = g * NBUF
        for b in range(NBUF):
            slot(j0 + b, b, True, True)
        return 0

    lax.fori_loop(1, N_GROUPS - 1, group_body, 0)

    j0 = (N_GROUPS - 1) * NBUF
    for b in range(NBUF):
        slot(j0 + b, b, True, b < NBUF - LOAD_AHEAD)
    for b in range(LOAD_AHEAD, NBUF):
        wait_scatter(b)

    pltpu.sync_copy(idx_hbm.at[pl.ds(row_base + N_MAIN * CHUNK, TAIL)],
                    idxt_v)
    rg_tail = rg_base + N_MAIN * RG_PER_CHUNK
    for i in range(TAIL // 8):
        pltpu.sync_copy(x_hbm.at[rg_tail + i, c],
                        tail_v.at[pl.ds(i * 8, 8)])
    pltpu.sync_copy(tail_v, acc_sh.at[idxt_v], add=True)
    pltpu.sync_copy(ones_v.at[pl.ds(0, TAIL)], cntacc_sh.at[idxt_v], add=True)

    plsc.subcore_barrier()

    seg_rg0 = s * (SEG_PER_TILE // 8)

    def store_strip(st, b, n=STRIP):
        rg = seg_rg0 + st * (STRIP // 8)
        for i in range(n // 8):
            pltpu.async_copy(rowsb[b].at[pl.ds(i * 8, 8)],
                             out_hbm.at[rg + i, c], out_sems[b])

    def wait_strip(b, n=STRIP):
        pltpu.make_async_copy(rowsb[b].at[pl.ds(0, n)],
                              dx_hbm.at[pl.ds(0, n)], out_sems[b]).wait()

    cntb = (cnt_v, cntb1)

    def start_read(st, b):
        pltpu.async_copy(acc_sh.at[pl.ds(seg0 + st * STRIP, STRIP)],
                         rowsb[b], rsems[b])
        pltpu.async_copy(cntacc_sh.at[pl.ds(seg0 + st * STRIP, STRIP)],
                         cntb[b], rsems[b])

    def wait_read(b):
        pltpu.make_async_copy(dx_hbm.at[pl.ds(0, CHUNK)], rowsb[b],
                              rsems[b]).wait()
        pltpu.make_async_copy(dx_hbm.at[pl.ds(0, 8)],
                              rowsb[b].at[pl.ds(0, 8)], rsems[b]).wait()

    def drain_store(st_done):
        b = st_done % 2
        if st_done < LAST_FULL_STRIPS:
            wait_strip(b)
        else:
            @pl.when(s < NS - 1)
            def _():
                wait_strip(b)

            if st_done == LAST_FULL_STRIPS:
                @pl.when(s == NS - 1)
                def _():
                    wait_strip(b, LAST_PART)

    start_read(0, 0)
    for st in range(N_STRIPS):
        b = st % 2
        wait_read(b)
        if st + 1 < N_STRIPS:
            if st >= 1:
                drain_store(st - 1)
            start_read(st + 1, 1 - b)

        def div_body(i, _):
            inv = ones16 / jnp.maximum(cntb[b][i, :], ones16)
            for j in range(COLS_PER_SC // L):
                rowsb[b][i, pl.ds(j * L, L)] = (
                    rowsb[b][i, pl.ds(j * L, L)] * inv)
            return 0

        lax.fori_loop(0, STRIP, div_body, 0)

        if st < LAST_FULL_STRIPS:
            store_strip(st, b)
        else:
            @pl.when(s < NS - 1)
            def _():
                store_strip(st, b)

            if st == LAST_FULL_STRIPS:
                @pl.when(s == NS - 1)
                def _():
                    store_strip(st, b, LAST_PART)

    drain_store(N_STRIPS - 2)
    drain_store(N_STRIPS - 1)


def _mean_pool(x, index, dx):
    run = pl.kernel(
        _body,
        out_type=jax.ShapeDtypeStruct((N_SEG // 8, NC, 8, COLS_PER_SC),
                                      jnp.float32),
        mesh=plsc.VectorSubcoreMesh(core_axis_name="c", subcore_axis_name="s"),
        compiler_params=pltpu.CompilerParams(use_tc_tiling_on_sc=False),
        scratch_types=[
            [pltpu.VMEM((CHUNK,), jnp.int32) for _ in range(NBUF)],
            [pltpu.VMEM((CHUNK, COLS_PER_SC), jnp.float32)
             for _ in range(NBUF)],
            pltpu.VMEM((TAIL,), jnp.int32),
            pltpu.VMEM((TAIL, COLS_PER_SC), jnp.float32),
            pltpu.VMEM((CHUNK, L), jnp.float32),
            pltpu.VMEM((STRIP, L), jnp.float32),
            pltpu.VMEM((STRIP, L), jnp.float32),
            pltpu.VMEM_SHARED((N_SEG_PAD, COLS_PER_SC), jnp.float32),
            pltpu.VMEM_SHARED((N_SEG_PAD, L), jnp.float32),
            [pltpu.SemaphoreType.DMA for _ in range(2 * NBUF)],
            [pltpu.SemaphoreType.DMA for _ in range(2)],
            [pltpu.SemaphoreType.DMA for _ in range(2)],
        ],
    )
    return run(x, index, dx)


@jax.jit
def kernel(x, index):
    x4 = x.reshape(NUM_ROWS // 8, 8, NC, COLS_PER_SC).transpose(0, 2, 1, 3)
    dx = jnp.zeros((CHUNK, COLS_PER_SC), jnp.float32)
    out4 = _mean_pool(x4, index.astype(jnp.int32), dx)
    return out4.transpose(0, 2, 1, 3).reshape(N_SEG, NUM_COLS)


# speedup vs baseline: 7.6278x; 1.0128x over previous
import jax
import jax.numpy as jnp
from jax import lax
from jax.experimental import pallas as pl
from jax.experimental.pallas import tpu as pltpu
from jax.experimental.pallas import tpu_sc as plsc

NUM_ROWS = 160000
NUM_COLS = 256
N_SEG = 10000
N_SEG_PAD = 10240

NC = 2
NS = 16
L = 16

COLS_PER_SC = NUM_COLS // NC
ROWS_PER_TILE = NUM_ROWS // NS
CHUNK = 32
N_MAIN = ROWS_PER_TILE // CHUNK
TAIL = ROWS_PER_TILE - N_MAIN * CHUNK
NBUF = 8
N_GROUPS = N_MAIN // NBUF
LOAD_AHEAD = 6
SEG_PER_TILE = N_SEG_PAD // NS
STRIP = CHUNK
N_STRIPS = SEG_PER_TILE // STRIP
LAST_SEGS = N_SEG - (NS - 1) * SEG_PER_TILE
LAST_FULL_STRIPS = LAST_SEGS // STRIP
LAST_PART = LAST_SEGS - LAST_FULL_STRIPS * STRIP


def _body(x_hbm, idx_hbm, dx_hbm, out_hbm,
          idxb, rowsb, idxt_v, tail_v, ones_v, cnt_v, cntb1,
          acc_sh, cntacc_sh, sems, out_sems, rsems):
    c = lax.axis_index("c")
    s = lax.axis_index("s")
    seg0 = s * SEG_PER_TILE
    row_base = s * ROWS_PER_TILE
    col0 = c * COLS_PER_SC

    zeros16 = jnp.zeros((L,), jnp.float32)
    ones16 = jnp.ones((L,), jnp.float32)

    def ones_body(i, _):
        ones_v[i, :] = ones16
        return 0

    lax.fori_loop(0, CHUNK, ones_body, 0)

    rg_base = s * (ROWS_PER_TILE // 8)
    RG_PER_CHUNK = CHUNK // 8

    def start_load(k, b):
        rg0 = rg_base + k * RG_PER_CHUNK
        for i in range(RG_PER_CHUNK):
            pltpu.async_copy(x_hbm.at[rg0 + i, c],
                             rowsb[b].at[pl.ds(i * 8, 8)], sems[b])
        pltpu.async_copy(idx_hbm.at[pl.ds(row_base + k * CHUNK, CHUNK)],
                         idxb[b], sems[b])

    def wait_load(b):
        pltpu.make_async_copy(dx_hbm.at[pl.ds(0, CHUNK)], rowsb[b],
                              sems[b]).wait()
        pltpu.make_async_copy(idx_hbm.at[pl.ds(row_base, CHUNK)],
                              idxb[b], sems[b]).wait()

    def start_scatter(k, b):
        pltpu.async_copy(rowsb[b], acc_sh.at[idxb[b]], sems[NBUF + b],
                         add=True)
        pltpu.async_copy(ones_v, cntacc_sh.at[idxb[b]], sems[NBUF + b],
                         add=True)

    def wait_scatter(b):
        pltpu.make_async_copy(rowsb[b], acc_sh.at[idxb[b]],
                              sems[NBUF + b]).wait()
        pltpu.make_async_copy(ones_v, cntacc_sh.at[idxb[b]],
                              sems[NBUF + b]).wait()

    for b in range(LOAD_AHEAD):
        start_load(b, b)

    def zero_body(i, _):
        for j in range(COLS_PER_SC // L):
            rowsb[NBUF - 1][i, pl.ds(j * L, L)] = zeros16
        cnt_v[i, :] = zeros16
        return 0

    lax.fori_loop(0, CHUNK, zero_body, 0)
    for st in range(N_STRIPS):
        pltpu.async_copy(rowsb[NBUF - 1],
                         acc_sh.at[pl.ds(seg0 + st * STRIP, STRIP)],
                         out_sems[0])
        pltpu.async_copy(cnt_v, cntacc_sh.at[pl.ds(seg0 + st * STRIP, STRIP)],
                         out_sems[1])
    for st in range(N_STRIPS):
        pltpu.make_async_copy(dx_hbm.at[pl.ds(0, CHUNK)], rowsb[NBUF - 1],
                              out_sems[0]).wait()
        pltpu.make_async_copy(dx_hbm.at[pl.ds(0, CHUNK // 8)],
                              rowsb[NBUF - 1].at[pl.ds(0, CHUNK // 8)],
                              out_sems[1]).wait()

    plsc.subcore_barrier()

    def slot(j, b, drain, load):
        wait_load(b)
        if drain:
            wait_scatter((b + LOAD_AHEAD) % NBUF)
        if load:
            start_load(j + LOAD_AHEAD, (b + LOAD_AHEAD) % NBUF)
        start_scatter(j, b)

    for b in range(NBUF):
        slot(b, b, b >= NBUF - LOAD_AHEAD, True)

    def group_body(g, _):
        j0 = g * NBUF
        for b in range(NBUF):
            slot(j0 + b, b, True, True)
        return 0

    lax.fori_loop(1, N_GROUPS - 1, group_body, 0)

    j0 = (N_GROUPS - 1) * NBUF
    for b in range(NBUF):
        slot(j0 + b, b, True, b < NBUF - LOAD_AHEAD)
    for b in range(LOAD_AHEAD, NBUF):
        wait_scatter(b)

    pltpu.sync_copy(idx_hbm.at[pl.ds(row_base + N_MAIN * CHUNK, TAIL)],
                    idxt_v)
    rg_tail = rg_base + N_MAIN * RG_PER_CHUNK
    for i in range(TAIL // 8):
        pltpu.sync_copy(x_hbm.at[rg_tail + i, c],
                        tail_v.at[pl.ds(i * 8, 8)])
    pltpu.sync_copy(tail_v, acc_sh.at[idxt_v], add=True)
    pltpu.sync_copy(ones_v.at[pl.ds(0, TAIL)], cntacc_sh.at[idxt_v], add=True)

    plsc.subcore_barrier()

    seg_rg0 = s * (SEG_PER_TILE // 8)

    def store_strip(st, b, n=STRIP):
        rg = seg_rg0 + st * (STRIP // 8)
        for i in range(n // 8):
            pltpu.async_copy(rowsb[b].at[pl.ds(i * 8, 8)],
                             out_hbm.at[rg + i, c], out_sems[b])

    def wait_strip(b, n=STRIP):
        pltpu.make_async_copy(rowsb[b].at[pl.ds(0, n)],
                              dx_hbm.at[pl.ds(0, n)], out_sems[b]).wait()

    cntb = (cnt_v, cntb1)

    def start_read(st, b):
        pltpu.async_copy(acc_sh.at[pl.ds(seg0 + st * STRIP, STRIP)],
                         rowsb[b], rsems[b])
        pltpu.async_copy(cntacc_sh.at[pl.ds(seg0 + st * STRIP, STRIP)],
                         cntb[b], rsems[b])

    def wait_read(b):
        pltpu.make_async_copy(dx_hbm.at[pl.ds(0, CHUNK)], rowsb[b],
                              rsems[b]).wait()
        pltpu.make_async_copy(dx_hbm.at[pl.ds(0, CHUNK // 8)],
                              rowsb[b].at[pl.ds(0, CHUNK // 8)], rsems[b]).wait()

    def drain_store(st_done):
        b = st_done % 2
        if st_done < LAST_FULL_STRIPS:
            wait_strip(b)
        else:
            @pl.when(s < NS - 1)
            def _():
                wait_strip(b)

            if st_done == LAST_FULL_STRIPS:
                @pl.when(s == NS - 1)
                def _():
                    wait_strip(b, LAST_PART)

    start_read(0, 0)
    for st in range(N_STRIPS):
        b = st % 2
        wait_read(b)
        if st + 1 < N_STRIPS:
            if st >= 1:
                drain_store(st - 1)
            start_read(st + 1, 1 - b)

        def div_body(i, _):
            inv = ones16 / jnp.maximum(cntb[b][i, :], ones16)
            for j in range(COLS_PER_SC // L):
                rowsb[b][i, pl.ds(j * L, L)] = (
                    rowsb[b][i, pl.ds(j * L, L)] * inv)
            return 0

        lax.fori_loop(0, STRIP, div_body, 0)

        if st < LAST_FULL_STRIPS:
            store_strip(st, b)
        else:
            @pl.when(s < NS - 1)
            def _():
                store_strip(st, b)

            if st == LAST_FULL_STRIPS:
                @pl.when(s == NS - 1)
                def _():
                    store_strip(st, b, LAST_PART)

    drain_store(N_STRIPS - 2)
    drain_store(N_STRIPS - 1)


def _mean_pool(x, index, dx):
    run = pl.kernel(
        _body,
        out_type=jax.ShapeDtypeStruct((N_SEG // 8, NC, 8, COLS_PER_SC),
                                      jnp.float32),
        mesh=plsc.VectorSubcoreMesh(core_axis_name="c", subcore_axis_name="s"),
        compiler_params=pltpu.CompilerParams(use_tc_tiling_on_sc=False),
        scratch_types=[
            [pltpu.VMEM((CHUNK,), jnp.int32) for _ in range(NBUF)],
            [pltpu.VMEM((CHUNK, COLS_PER_SC), jnp.float32)
             for _ in range(NBUF)],
            pltpu.VMEM((TAIL,), jnp.int32),
            pltpu.VMEM((TAIL, COLS_PER_SC), jnp.float32),
            pltpu.VMEM((CHUNK, L), jnp.float32),
            pltpu.VMEM((STRIP, L), jnp.float32),
            pltpu.VMEM((STRIP, L), jnp.float32),
            pltpu.VMEM_SHARED((N_SEG_PAD, COLS_PER_SC), jnp.float32),
            pltpu.VMEM_SHARED((N_SEG_PAD, L), jnp.float32),
            [pltpu.SemaphoreType.DMA for _ in range(2 * NBUF)],
            [pltpu.SemaphoreType.DMA for _ in range(2)],
            [pltpu.SemaphoreType.DMA for _ in range(2)],
        ],
    )
    return run(x, index, dx)


@jax.jit
def kernel(x, index):
    x4 = x.reshape(NUM_ROWS // 8, 8, NC, COLS_PER_SC).transpose(0, 2, 1, 3)
    dx = jnp.zeros((CHUNK, COLS_PER_SC), jnp.float32)
    out4 = _mean_pool(x4, index.astype(jnp.int32), dx)
    return out4.transpose(0, 2, 1, 3).reshape(N_SEG, NUM_COLS)
